# Initial kernel scaffold; baseline (speedup 1.0000x reference)
import jax
import jax.numpy as jnp
from jax.experimental import pallas as pl


def kernel(t, dim, index, src):
    raise NotImplementedError("write your pallas kernel here")


# speedup vs baseline: 23.1556x; 23.1556x over previous
import functools

import jax
import jax.numpy as jnp
from jax import lax
from jax.experimental import pallas as pl
from jax.experimental.pallas import tpu as pltpu
from jax.experimental.pallas import tpu_sc as plsc

_NUM_CORES = 2
_NUM_SUBCORES = 16
_NW = _NUM_CORES * _NUM_SUBCORES
_LANES = 16
_CHUNK = 8192


def _sc_scatter_cols(t_cm, idx_cm, src_cm, m, d, b):
    cols_per_w = d // _NW
    n_chunks = b // _CHUNK
    mesh = plsc.VectorSubcoreMesh(core_axis_name="c", subcore_axis_name="s")

    @functools.partial(
        pl.kernel,
        mesh=mesh,
        out_type=jax.ShapeDtypeStruct((d, m), jnp.float32),
        scratch_types=[
            pltpu.VMEM((m,), jnp.float32),
            pltpu.VMEM((b,), jnp.int32),
            pltpu.VMEM((_CHUNK,), jnp.float32),
        ],
        compiler_params=pltpu.CompilerParams(needs_layout_passes=False),
    )
    def k(t_hbm, idx_hbm, src_hbm, out_hbm, acc_v, idx_v, src_v):
        wid = lax.axis_index("s") * _NUM_CORES + lax.axis_index("c")

        def per_col(ci, carry):
            j = wid * cols_per_w + ci
            pltpu.sync_copy(t_hbm.at[j], acc_v)
            pltpu.sync_copy(idx_hbm.at[j], idx_v)

            def per_chunk(ch, carry2):
                pltpu.sync_copy(
                    src_hbm.at[j, pl.ds(ch * _CHUNK, _CHUNK)], src_v
                )

                def upd(u, carry3):
                    iv = idx_v[pl.ds(ch * _CHUNK + u * _LANES, _LANES)]
                    sv = src_v[pl.ds(u * _LANES, _LANES)]
                    plsc.addupdate_scatter(acc_v, [iv], sv)
                    return carry3

                return lax.fori_loop(0, _CHUNK // _LANES, upd, carry2, unroll=8)

            lax.fori_loop(0, n_chunks, per_chunk, 0)
            pltpu.sync_copy(acc_v, out_hbm.at[j])
            return carry

        lax.fori_loop(0, cols_per_w, per_col, 0)

    return k(t_cm, idx_cm, src_cm)


def _tc_transpose(x_cm, m, d):
    bm = 1024

    def body(x_ref, o_ref):
        o_ref[...] = x_ref[...].T

    return pl.pallas_call(
        body,
        grid=(pl.cdiv(m, bm),),
        in_specs=[pl.BlockSpec((d, bm), lambda i: (0, i))],
        out_specs=pl.BlockSpec((bm, d), lambda i: (i, 0)),
        out_shape=jax.ShapeDtypeStruct((m, d), jnp.float32),
    )(x_cm)


def kernel(t, dim, index, src):
    del dim
    m, d = t.shape
    b = src.shape[0]
    t_cm = t.T
    idx_cm = index.astype(jnp.int32).T
    src_cm = src.T
    out_cm = _sc_scatter_cols(t_cm, idx_cm, src_cm, m, d, b)
    return _tc_transpose(out_cm, m, d)
